# baseline (speedup 1.0000x reference)
import jax
import jax.numpy as jnp
from jax import lax
from jax.experimental import pallas as pl
from jax.experimental.pallas import tpu as pltpu
from jax.experimental.pallas import tpu_sc as plsc

L = 16
NC, NS = 2, 16
NW = NC * NS
NBUF = 4
H = 524288


def _sc_pool_kernel(b, seq, d, vpacked):
    rows_per_w = b // NW
    dchunks = d // L
    inv = 1.0 / seq

    def body(p_hbm, h_hbm, tab_hbm, outp_hbm, outh_hbm,
             idxp_v, idxh_v, b0, b1, b2, b3, pool_v, s0, s1, s2, s3):
        cid = lax.axis_index("c")
        sid = lax.axis_index("s")
        wid = sid * NC + cid
        base = wid * rows_per_w

        bufs = (b0, b1, b2, b3)
        sems = (s0, s1, s2, s3)

        pltpu.sync_copy(p_hbm.at[pl.ds(base, rows_per_w)], idxp_v)
        pltpu.sync_copy(h_hbm.at[pl.ds(base, rows_per_w)], idxh_v)

        zeros = tuple(jnp.zeros((L,), jnp.float32) for _ in range(dchunks))

        for idx_v, out_hbm in ((idxp_v, outp_hbm), (idxh_v, outh_hbm)):
            for k in range(NBUF):
                pltpu.async_copy(tab_hbm.at[idx_v.at[k]], bufs[k], sems[k])

            def step(i, carry):
                for k in range(NBUF):
                    g = NBUF * i + k
                    pltpu.make_async_copy(
                        tab_hbm.at[idx_v.at[g]], bufs[k], sems[k]).wait()

                    def s_body(s, accs, _buf=bufs[k]):
                        return tuple(accs[c] + _buf[s, pl.ds(c * L, L)]
                                     for c in range(dchunks))
                    accs = lax.fori_loop(0, seq, s_body, zeros)
                    for c in range(dchunks):
                        pool_v[g, pl.ds(c * L, L)] = accs[c] * inv

                    @pl.when(g + NBUF < rows_per_w)
                    def _():
                        pltpu.async_copy(
                            tab_hbm.at[idx_v.at[g + NBUF]], bufs[k], sems[k])
                return carry

            lax.fori_loop(0, rows_per_w // NBUF, step, 0)
            pltpu.sync_copy(pool_v, out_hbm.at[pl.ds(base, rows_per_w)])

    return pl.kernel(
        body,
        out_type=[jax.ShapeDtypeStruct((b, d), jnp.float32),
                  jax.ShapeDtypeStruct((b, d), jnp.float32)],
        mesh=plsc.VectorSubcoreMesh(core_axis_name="c", subcore_axis_name="s",
                                    num_cores=NC, num_subcores=NS),
        scratch_types=(
            [pltpu.VMEM((rows_per_w, seq), jnp.int32)] * 2
            + [pltpu.VMEM((seq, d), jnp.float32)] * NBUF
            + [pltpu.VMEM((rows_per_w, d), jnp.float32)]
            + [pltpu.SemaphoreType.DMA] * NBUF
        ),
        compiler_params=pltpu.CompilerParams(use_tc_tiling_on_sc=False),
    )


def _repack_body(xa_ref, xb_ref, eye_ref, o_ref):
    d = xa_ref.shape[0]
    dn = (((0,), (0,)), ((), ()))
    o_ref[:, 0:d] = lax.dot_general(xa_ref[...], eye_ref[...], dn,
                                    preferred_element_type=jnp.float32)
    o_ref[:, d:] = lax.dot_general(xb_ref[...], eye_ref[...], dn,
                                   preferred_element_type=jnp.float32)


def _repack(tab_t, v, d):
    bv = 4096
    edge = v // bv
    return pl.pallas_call(
        _repack_body,
        grid=(H // bv,),
        in_specs=[
            pl.BlockSpec((d, bv), lambda i: (0, i)),
            pl.BlockSpec((d, bv), lambda i: (0, jnp.minimum(i + H // bv, edge))),
            pl.BlockSpec((d, d), lambda i: (0, 0)),
        ],
        out_specs=pl.BlockSpec((bv, 2 * d), lambda i: (i, 0)),
        out_shape=jax.ShapeDtypeStruct((H, 2 * d), jnp.float32),
    )(tab_t, tab_t, jnp.eye(d, dtype=jnp.float32))


def _linear_body(xp_ref, xh_ref, w_ref, b_ref, o_ref):
    d = xp_ref.shape[1]
    o_ref[...] = (
        jnp.dot(xp_ref[...], w_ref[0:d, :], preferred_element_type=jnp.float32)
        + jnp.dot(xh_ref[...], w_ref[d:2 * d, :], preferred_element_type=jnp.float32)
        + b_ref[...]
    )


def kernel(premise, hypothesis, emb_table, fc_w, fc_b):
    b, seq = premise.shape
    _, d = emb_table.shape
    odim = fc_w.shape[0]

    v = emb_table.shape[0]
    tab_c = _repack(jnp.swapaxes(emb_table, 0, 1), v, d)
    tab_lin = jnp.reshape(tab_c, (2 * H, d))

    def remap(ix):
        ix = ix.astype(jnp.int32)
        return jnp.where(ix < H, 2 * ix, 2 * (ix - H) + 1).astype(jnp.int32)

    pooled_p, pooled_h = _sc_pool_kernel(b, seq, d, 2 * H)(
        remap(premise), remap(hypothesis), tab_lin)

    wpad = jnp.zeros((2 * d, 128), jnp.float32).at[:, :odim].set(fc_w.T)
    bpad = jnp.zeros((1, 128), jnp.float32).at[0, :odim].set(fc_b)

    bm = 512
    out_pad = pl.pallas_call(
        _linear_body,
        grid=(b // bm,),
        in_specs=[
            pl.BlockSpec((bm, d), lambda i: (i, 0)),
            pl.BlockSpec((bm, d), lambda i: (i, 0)),
            pl.BlockSpec((2 * d, 128), lambda i: (0, 0)),
            pl.BlockSpec((1, 128), lambda i: (0, 0)),
        ],
        out_specs=pl.BlockSpec((bm, 128), lambda i: (i, 0)),
        out_shape=jax.ShapeDtypeStruct((b, 128), jnp.float32),
    )(pooled_p, pooled_h, wpad, bpad)

    return out_pad[:, :odim]


# speedup vs baseline: 1.1075x; 1.1075x over previous
import jax
import jax.numpy as jnp
from jax import lax
from jax.experimental import pallas as pl
from jax.experimental.pallas import tpu as pltpu
from jax.experimental.pallas import tpu_sc as plsc

L = 16
NC, NS = 2, 16
NW = NC * NS
NBUF = 4
H = 524288


def _sc_pool_kernel(b, seq, d, vpacked):
    rows_per_w = b // NW
    dchunks = d // L
    inv = 1.0 / seq

    def body(p_hbm, h_hbm, tab_hbm, outp_hbm, outh_hbm,
             idxp_v, idxh_v, b0, b1, b2, b3, pool_v, s0, s1, s2, s3):
        cid = lax.axis_index("c")
        sid = lax.axis_index("s")
        wid = sid * NC + cid
        base = wid * rows_per_w

        bufs = (b0, b1, b2, b3)
        sems = (s0, s1, s2, s3)

        pltpu.sync_copy(p_hbm.at[pl.ds(base, rows_per_w)], idxp_v)
        pltpu.sync_copy(h_hbm.at[pl.ds(base, rows_per_w)], idxh_v)

        zeros = tuple(jnp.zeros((L,), jnp.float32) for _ in range(dchunks))

        for idx_v, out_hbm in ((idxp_v, outp_hbm), (idxh_v, outh_hbm)):
            for k in range(NBUF):
                pltpu.async_copy(tab_hbm.at[idx_v.at[k]], bufs[k], sems[k])

            def step(i, carry):
                for k in range(NBUF):
                    g = NBUF * i + k
                    pltpu.make_async_copy(
                        tab_hbm.at[idx_v.at[g]], bufs[k], sems[k]).wait()

                    def s_body(s, accs, _buf=bufs[k]):
                        return tuple(accs[c] + _buf[s, pl.ds(c * L, L)]
                                     for c in range(dchunks))
                    accs = lax.fori_loop(0, seq, s_body, zeros)
                    for c in range(dchunks):
                        pool_v[g, pl.ds(c * L, L)] = accs[c] * inv

                    @pl.when(g + NBUF < rows_per_w)
                    def _():
                        pltpu.async_copy(
                            tab_hbm.at[idx_v.at[g + NBUF]], bufs[k], sems[k])
                return carry

            lax.fori_loop(0, rows_per_w // NBUF, step, 0)
            pltpu.sync_copy(pool_v, out_hbm.at[pl.ds(base, rows_per_w)])

    return pl.kernel(
        body,
        out_type=[jax.ShapeDtypeStruct((b, d), jnp.float32),
                  jax.ShapeDtypeStruct((b, d), jnp.float32)],
        mesh=plsc.VectorSubcoreMesh(core_axis_name="c", subcore_axis_name="s",
                                    num_cores=NC, num_subcores=NS),
        scratch_types=(
            [pltpu.VMEM((rows_per_w, seq), jnp.int32)] * 2
            + [pltpu.VMEM((seq, d), jnp.float32)] * NBUF
            + [pltpu.VMEM((rows_per_w, d), jnp.float32)]
            + [pltpu.SemaphoreType.DMA] * NBUF
        ),
        compiler_params=pltpu.CompilerParams(use_tc_tiling_on_sc=False),
    )


def _repack_body(xa_ref, xb_ref, o_ref):
    d = xa_ref.shape[0]
    o_ref[:, 0:d] = jnp.swapaxes(xa_ref[...], 0, 1)
    o_ref[:, d:] = jnp.swapaxes(xb_ref[...], 0, 1)


def _repack(tab_t, v, d):
    bv = 8192
    edge = v // bv
    return pl.pallas_call(
        _repack_body,
        grid=(H // bv,),
        in_specs=[
            pl.BlockSpec((d, bv), lambda i: (0, i)),
            pl.BlockSpec((d, bv), lambda i: (0, jnp.minimum(i + H // bv, edge))),
        ],
        out_specs=pl.BlockSpec((bv, 2 * d), lambda i: (i, 0)),
        out_shape=jax.ShapeDtypeStruct((H, 2 * d), jnp.float32),
        compiler_params=pltpu.CompilerParams(vmem_limit_bytes=100 * 1024 * 1024),
    )(tab_t, tab_t)


def _linear_body(xp_ref, xh_ref, w_ref, b_ref, o_ref):
    d = xp_ref.shape[1]
    o_ref[...] = (
        jnp.dot(xp_ref[...], w_ref[0:d, :], preferred_element_type=jnp.float32)
        + jnp.dot(xh_ref[...], w_ref[d:2 * d, :], preferred_element_type=jnp.float32)
        + b_ref[...]
    )


def kernel(premise, hypothesis, emb_table, fc_w, fc_b):
    b, seq = premise.shape
    _, d = emb_table.shape
    odim = fc_w.shape[0]

    v = emb_table.shape[0]
    tab_c = _repack(jnp.swapaxes(emb_table, 0, 1), v, d)
    tab_lin = jnp.reshape(tab_c, (2 * H, d))

    def remap(ix):
        ix = ix.astype(jnp.int32)
        return jnp.where(ix < H, 2 * ix, 2 * (ix - H) + 1).astype(jnp.int32)

    pooled_p, pooled_h = _sc_pool_kernel(b, seq, d, 2 * H)(
        remap(premise), remap(hypothesis), tab_lin)

    wpad = jnp.zeros((2 * d, 128), jnp.float32).at[:, :odim].set(fc_w.T)
    bpad = jnp.zeros((1, 128), jnp.float32).at[0, :odim].set(fc_b)

    bm = 512
    out_pad = pl.pallas_call(
        _linear_body,
        grid=(b // bm,),
        in_specs=[
            pl.BlockSpec((bm, d), lambda i: (i, 0)),
            pl.BlockSpec((bm, d), lambda i: (i, 0)),
            pl.BlockSpec((2 * d, 128), lambda i: (0, 0)),
            pl.BlockSpec((1, 128), lambda i: (0, 0)),
        ],
        out_specs=pl.BlockSpec((bm, 128), lambda i: (i, 0)),
        out_shape=jax.ShapeDtypeStruct((b, 128), jnp.float32),
    )(pooled_p, pooled_h, wpad, bpad)

    return out_pad[:, :odim]


# speedup vs baseline: 1.1558x; 1.0436x over previous
import jax
import jax.numpy as jnp
from jax import lax
from jax.experimental import pallas as pl
from jax.experimental.pallas import tpu as pltpu
from jax.experimental.pallas import tpu_sc as plsc

L = 16
NC, NS = 2, 16
NW = NC * NS
NBUF = 4
H = 524288


def _sc_pool_kernel(b, seq, d, vpacked):
    rows_per_w = b // NW
    dchunks = d // L
    inv = 1.0 / seq

    def body(p_hbm, h_hbm, tab_hbm, outp_hbm, outh_hbm,
             idxp_v, idxh_v, b0, b1, b2, b3, pool_v, s0, s1, s2, s3):
        cid = lax.axis_index("c")
        sid = lax.axis_index("s")
        wid = sid * NC + cid
        base = wid * rows_per_w

        bufs = (b0, b1, b2, b3)
        sems = (s0, s1, s2, s3)

        pltpu.sync_copy(p_hbm.at[pl.ds(base, rows_per_w)], idxp_v)
        pltpu.sync_copy(h_hbm.at[pl.ds(base, rows_per_w)], idxh_v)

        zeros = tuple(jnp.zeros((L,), jnp.float32) for _ in range(dchunks))

        for idx_v, out_hbm in ((idxp_v, outp_hbm), (idxh_v, outh_hbm)):
            for k in range(NBUF):
                pltpu.async_copy(tab_hbm.at[idx_v.at[k]], bufs[k], sems[k])

            def step(i, carry):
                for k in range(NBUF):
                    g = NBUF * i + k
                    pltpu.make_async_copy(
                        tab_hbm.at[idx_v.at[g]], bufs[k], sems[k]).wait()

                    def s_body(s, accs, _buf=bufs[k]):
                        return tuple(accs[c] + _buf[s, pl.ds(c * L, L)]
                                     for c in range(dchunks))
                    accs = lax.fori_loop(0, seq, s_body, zeros)
                    for c in range(dchunks):
                        pool_v[g, pl.ds(c * L, L)] = accs[c] * inv

                    @pl.when(g + NBUF < rows_per_w)
                    def _():
                        pltpu.async_copy(
                            tab_hbm.at[idx_v.at[g + NBUF]], bufs[k], sems[k])
                return carry

            lax.fori_loop(0, rows_per_w // NBUF, step, 0)
            pltpu.sync_copy(pool_v, out_hbm.at[pl.ds(base, rows_per_w)])

    return pl.kernel(
        body,
        out_type=[jax.ShapeDtypeStruct((b, d), jnp.float32),
                  jax.ShapeDtypeStruct((b, d), jnp.float32)],
        mesh=plsc.VectorSubcoreMesh(core_axis_name="c", subcore_axis_name="s",
                                    num_cores=NC, num_subcores=NS),
        scratch_types=(
            [pltpu.VMEM((rows_per_w, seq), jnp.int32)] * 2
            + [pltpu.VMEM((seq, d), jnp.float32)] * NBUF
            + [pltpu.VMEM((rows_per_w, d), jnp.float32)]
            + [pltpu.SemaphoreType.DMA] * NBUF
        ),
        compiler_params=pltpu.CompilerParams(use_tc_tiling_on_sc=False),
    )


def _repack_body(xa_ref, xb_ref, o_ref):
    d = xa_ref.shape[0]
    o_ref[:, 0:d] = jnp.swapaxes(xa_ref[...], 0, 1)
    o_ref[:, d:] = jnp.swapaxes(xb_ref[...], 0, 1)


def _repack(tab_t, v, d):
    bv = 16384
    edge = v // bv
    return pl.pallas_call(
        _repack_body,
        grid=(H // bv,),
        in_specs=[
            pl.BlockSpec((d, bv), lambda i: (0, i)),
            pl.BlockSpec((d, bv), lambda i: (0, jnp.minimum(i + H // bv, edge))),
        ],
        out_specs=pl.BlockSpec((bv, 2 * d), lambda i: (i, 0)),
        out_shape=jax.ShapeDtypeStruct((H, 2 * d), jnp.float32),
        compiler_params=pltpu.CompilerParams(vmem_limit_bytes=100 * 1024 * 1024),
    )(tab_t, tab_t)


def _linear_body(xp_ref, xh_ref, w_ref, b_ref, o_ref):
    d = xp_ref.shape[1]
    o_ref[...] = (
        jnp.dot(xp_ref[...], w_ref[0:d, :], preferred_element_type=jnp.float32)
        + jnp.dot(xh_ref[...], w_ref[d:2 * d, :], preferred_element_type=jnp.float32)
        + b_ref[...]
    )


def kernel(premise, hypothesis, emb_table, fc_w, fc_b):
    b, seq = premise.shape
    _, d = emb_table.shape
    odim = fc_w.shape[0]

    v = emb_table.shape[0]
    tab_c = _repack(jnp.swapaxes(emb_table, 0, 1), v, d)
    tab_lin = jnp.reshape(tab_c, (2 * H, d))

    def remap(ix):
        ix = ix.astype(jnp.int32)
        return jnp.where(ix < H, 2 * ix, 2 * (ix - H) + 1).astype(jnp.int32)

    pooled_p, pooled_h = _sc_pool_kernel(b, seq, d, 2 * H)(
        remap(premise), remap(hypothesis), tab_lin)

    wpad = jnp.zeros((2 * d, 128), jnp.float32).at[:, :odim].set(fc_w.T)
    bpad = jnp.zeros((1, 128), jnp.float32).at[0, :odim].set(fc_b)

    bm = 512
    out_pad = pl.pallas_call(
        _linear_body,
        grid=(b // bm,),
        in_specs=[
            pl.BlockSpec((bm, d), lambda i: (i, 0)),
            pl.BlockSpec((bm, d), lambda i: (i, 0)),
            pl.BlockSpec((2 * d, 128), lambda i: (0, 0)),
            pl.BlockSpec((1, 128), lambda i: (0, 0)),
        ],
        out_specs=pl.BlockSpec((bm, 128), lambda i: (i, 0)),
        out_shape=jax.ShapeDtypeStruct((b, 128), jnp.float32),
    )(pooled_p, pooled_h, wpad, bpad)

    return out_pad[:, :odim]
